# Initial kernel scaffold; baseline (speedup 1.0000x reference)
import jax
import jax.numpy as jnp
from jax.experimental import pallas as pl


def kernel(hidden_states, attention_mask, Ws1, bs1, Ws2, bs2, Wp1, bp1, Wp2, bp2, gamma, beta):
    raise NotImplementedError("write your pallas kernel here")


# speedup vs baseline: 1.3832x; 1.3832x over previous
import functools
import math

import jax
import jax.numpy as jnp
from jax import lax
from jax.experimental import pallas as pl
from jax.experimental.pallas import tpu as pltpu
from jax.experimental.pallas import tpu_sc as plsc

D_MODEL = 768
D_SCORE = 192
D_DETAIL = 384
B = 4
S = 8192
K = 32

S_BLK = 2048
N_BLKS = (B * S) // S_BLK

MASK_BASE = -3.0e38
MASK_STEP = 4.0e33
MASK_THRESH = -1.0e38
NEG_INF = float("-inf")

_SQRT2 = math.sqrt(2.0)


def _gelu_exact(x):
    return 0.5 * x * (1.0 + lax.erf(x / _SQRT2))


def _scorer_body(h_ref, m_ref, w1_ref, b1_ref, w2_ref, b2_ref, out_ref):
    i = pl.program_id(0)
    h = h_ref[0]
    s = jnp.dot(h, w1_ref[...], preferred_element_type=jnp.float32)
    s = _gelu_exact(s + b1_ref[...])
    sc = jnp.dot(s, w2_ref[...], preferred_element_type=jnp.float32)
    sc = sc + b2_ref[0, 0]
    j = lax.rem(i, S // S_BLK)
    pos = (lax.broadcasted_iota(jnp.int32, (S_BLK, 1), 0) + S_BLK * j).astype(jnp.float32)
    masked_val = MASK_BASE - pos * MASK_STEP
    out_ref[...] = jnp.where(m_ref[...] == 0.0, masked_val, sc)


def _run_scorer(hidden, attention_mask, Ws1, bs1, Ws2, bs2):
    h3 = hidden.reshape(N_BLKS, S_BLK, D_MODEL)
    mf = attention_mask.astype(jnp.float32).reshape(B * S, 1)
    scores = pl.pallas_call(
        _scorer_body,
        grid=(N_BLKS,),
        in_specs=[
            pl.BlockSpec((1, S_BLK, D_MODEL), lambda i: (i, 0, 0)),
            pl.BlockSpec((S_BLK, 1), lambda i: (i, 0)),
            pl.BlockSpec((D_MODEL, D_SCORE), lambda i: (0, 0)),
            pl.BlockSpec((1, D_SCORE), lambda i: (0, 0)),
            pl.BlockSpec((D_SCORE, 1), lambda i: (0, 0)),
            pl.BlockSpec(memory_space=pltpu.SMEM),
        ],
        out_specs=pl.BlockSpec((S_BLK, 1), lambda i: (i, 0)),
        out_shape=jax.ShapeDtypeStruct((B * S, 1), jnp.float32),
    )(h3, mf, Ws1, bs1.reshape(1, D_SCORE), Ws2, bs2.reshape(1, 1))
    return scores.reshape(B * S)


CHUNK = 1024
SUBS_PER_BATCH = 8
G = 8
GROUP = CHUNK // G
L = 16

_LANE_IOTA = None


def _sc_body(scores_hbm, hidden_hbm, sel_out, mask_out,
             buf, l1v, l1i, locv, loci, mrgv, mrgi, finv, fini, dmask,
             gidx, rows, shv, shi, shfi, sem):
    c = lax.axis_index("c")
    s = lax.axis_index("s")
    b_local = s // SUBS_PER_BATCH
    chunk = lax.rem(s, SUBS_PER_BATCH)
    b = 2 * c + b_local
    offset = b * S + chunk * CHUNK

    lane = lax.broadcasted_iota(jnp.int32, (L,), 0)
    lane0 = lane == 0

    pltpu.sync_copy(scores_hbm.at[pl.ds(offset, CHUNK)], buf)

    for g in range(G):
        base = g * GROUP
        val = buf[pl.ds(base, L)]
        idx = lane + (offset + base)
        for r in range(1, GROUP // L):
            v2 = buf[pl.ds(base + r * L, L)]
            i2 = lane + (offset + base + r * L)
            take = v2 > val
            val = jnp.where(take, v2, val)
            idx = jnp.where(take, i2, idx)
        l1v[pl.ds(g * L, L)] = val
        l1i[pl.ds(g * L, L)] = idx

    def _extract(t, _):
        val = l1v[pl.ds(0, L)]
        idx = l1i[pl.ds(0, L)]
        for g in range(1, G):
            v2 = l1v[pl.ds(g * L, L)]
            i2 = l1i[pl.ds(g * L, L)]
            take = v2 > val
            val = jnp.where(take, v2, val)
            idx = jnp.where(take, i2, idx)
        m = jnp.max(val)
        win = jnp.min(jnp.where(val == m, idx, jnp.int32(2**31 - 1)))
        plsc.store_scatter(locv, [jnp.full((L,), t, jnp.int32)],
                           jnp.full((L,), m, jnp.float32), mask=lane0)
        plsc.store_scatter(loci, [jnp.full((L,), t, jnp.int32)],
                           jnp.full((L,), win, jnp.int32), mask=lane0)
        p = win - offset
        plsc.store_scatter(buf, [jnp.full((L,), p, jnp.int32)],
                           jnp.full((L,), NEG_INF, jnp.float32), mask=lane0)
        g = p // GROUP
        gbase = g * GROUP
        nval = buf[pl.ds(gbase, L)]
        nidx = lane + (offset + gbase)
        for r in range(1, GROUP // L):
            v2 = buf[pl.ds(gbase + r * L, L)]
            i2 = lane + (offset + gbase + r * L)
            take = v2 > nval
            nval = jnp.where(take, v2, nval)
            nidx = jnp.where(take, i2, nidx)
        l1v[pl.ds(g * L, L)] = nval
        l1i[pl.ds(g * L, L)] = nidx
        return 0

    lax.fori_loop(0, K, _extract, 0)

    pltpu.sync_copy(locv, shv.at[pl.ds(s * K, K)])
    pltpu.sync_copy(loci, shi.at[pl.ds(s * K, K)])
    plsc.subcore_barrier()

    @pl.when(lax.rem(s, SUBS_PER_BATCH) == 0)
    def _merge():
        s0 = b_local * SUBS_PER_BATCH
        pltpu.sync_copy(shv.at[pl.ds(s0 * K, SUBS_PER_BATCH * K)], mrgv)
        pltpu.sync_copy(shi.at[pl.ds(s0 * K, SUBS_PER_BATCH * K)], mrgi)

        n_cells = SUBS_PER_BATCH * K // L

        def _mext(t, _):
            val = mrgv[pl.ds(0, L)]
            idx = mrgi[pl.ds(0, L)]
            pos = lane
            for k in range(1, n_cells):
                v2 = mrgv[pl.ds(k * L, L)]
                i2 = mrgi[pl.ds(k * L, L)]
                p2 = lane + k * L
                take = (v2 > val) | ((v2 == val) & (i2 < idx))
                val = jnp.where(take, v2, val)
                idx = jnp.where(take, i2, idx)
                pos = jnp.where(take, p2, pos)
            m = jnp.max(val)
            big = jnp.int32(2**31 - 1)
            win = jnp.min(jnp.where(val == m, idx, big))
            wpos = jnp.min(jnp.where((val == m) & (idx == win), pos, big))
            plsc.store_scatter(finv, [jnp.full((L,), t, jnp.int32)],
                               jnp.full((L,), m, jnp.float32), mask=lane0)
            plsc.store_scatter(fini, [jnp.full((L,), t, jnp.int32)],
                               jnp.full((L,), win, jnp.int32), mask=lane0)
            plsc.store_scatter(mrgv, [jnp.full((L,), wpos, jnp.int32)],
                               jnp.full((L,), NEG_INF, jnp.float32), mask=lane0)
            return 0

        lax.fori_loop(0, K, _mext, 0)

        for h in range(K // L):
            v = finv[pl.ds(h * L, L)]
            dmask[pl.ds(h * L, L)] = jnp.where(v > MASK_THRESH,
                                               jnp.float32(1.0), jnp.float32(0.0))
        pltpu.sync_copy(dmask, mask_out.at[b])
        pltpu.sync_copy(fini, shfi.at[pl.ds(b_local * K, K)])

    plsc.subcore_barrier()

    @pl.when(lax.rem(s, SUBS_PER_BATCH) < 4)
    def _gather():
        r0 = lax.rem(s, SUBS_PER_BATCH) * 8
        pltpu.sync_copy(shfi.at[pl.ds(b_local * K + r0, 8)], gidx)
        pltpu.async_copy(hidden_hbm.at[gidx], rows, sem).wait()
        pltpu.sync_copy(rows, sel_out.at[pl.ds(b * K + r0, 8)])


def _run_sc_topk_gather(scores_flat, hidden_flat):
    mesh = plsc.VectorSubcoreMesh(core_axis_name="c", subcore_axis_name="s",
                                  num_cores=2, num_subcores=16)
    f32 = jnp.float32
    i32 = jnp.int32
    kern = functools.partial(
        pl.kernel,
        out_type=[
            jax.ShapeDtypeStruct((B * K, D_MODEL), f32),
            jax.ShapeDtypeStruct((B, K), f32),
        ],
        mesh=mesh,
        compiler_params=pltpu.CompilerParams(needs_layout_passes=False),
        scratch_types=[
            pltpu.VMEM((CHUNK,), f32),
            pltpu.VMEM((G * L,), f32),
            pltpu.VMEM((G * L,), i32),
            pltpu.VMEM((K,), f32),
            pltpu.VMEM((K,), i32),
            pltpu.VMEM((SUBS_PER_BATCH * K,), f32),
            pltpu.VMEM((SUBS_PER_BATCH * K,), i32),
            pltpu.VMEM((K,), f32),
            pltpu.VMEM((K,), i32),
            pltpu.VMEM((K,), f32),
            pltpu.VMEM((8,), i32),
            pltpu.VMEM((8, D_MODEL), f32),
            pltpu.VMEM_SHARED((16 * K,), f32),
            pltpu.VMEM_SHARED((16 * K,), i32),
            pltpu.VMEM_SHARED((2 * K,), i32),
            pltpu.SemaphoreType.DMA,
        ],
    )(_sc_body)
    return kern(scores_flat, hidden_flat)


def _proj_body(x_ref, w1_ref, b1_ref, w2_ref, b2_ref, g_ref, bt_ref, out_ref):
    x = x_ref[...]
    h = jnp.dot(x, w1_ref[...], preferred_element_type=jnp.float32) + b1_ref[...]
    h = _gelu_exact(h)
    d = jnp.dot(h, w2_ref[...], preferred_element_type=jnp.float32) + b2_ref[...]
    mu = jnp.mean(d, axis=-1, keepdims=True)
    var = jnp.mean((d - mu) ** 2, axis=-1, keepdims=True)
    out_ref[...] = (d - mu) / jnp.sqrt(var + 1e-5) * g_ref[...] + bt_ref[...]


def _run_proj(selected, Wp1, bp1, Wp2, bp2, gamma, beta):
    return pl.pallas_call(
        _proj_body,
        out_shape=jax.ShapeDtypeStruct((B * K, D_DETAIL), jnp.float32),
    )(selected, Wp1, bp1.reshape(1, D_DETAIL), Wp2, bp2.reshape(1, D_DETAIL),
      gamma.reshape(1, D_DETAIL), beta.reshape(1, D_DETAIL))


def kernel(hidden_states, attention_mask, Ws1, bs1, Ws2, bs2,
           Wp1, bp1, Wp2, bp2, gamma, beta):
    scores = _run_scorer(hidden_states, attention_mask, Ws1, bs1, Ws2, bs2)
    hidden_flat = hidden_states.reshape(B * S, D_MODEL)
    selected, detail_mask = _run_sc_topk_gather(scores, hidden_flat)
    d = _run_proj(selected, Wp1, bp1, Wp2, bp2, gamma, beta)
    return d.reshape(B, K, D_DETAIL), detail_mask
